# baseline (speedup 1.0000x reference)
import jax
import jax.numpy as jnp
from jax import lax
from jax.experimental import pallas as pl
from jax.experimental.pallas import tpu as pltpu
from jax.experimental.pallas import tpu_sc as plsc

_VOCAB = 100000
_DEPTH = 4
_NCHILD = 64
_B, _T = 4, 32
_ROWS = _B * _T
_UPD = _DEPTH * _NCHILD
_NW = 32
_ROWS_PER_W = _ROWS // _NW
_FILL_UNROLL = 25
_FILL_STEPS = _VOCAB // (16 * _FILL_UNROLL)


def _sc_body(key_hbm, val_hbm, b_hbm, out_hbm,
             row_v, key_v, val_v, b_v):
    wid = lax.axis_index("s") * 2 + lax.axis_index("c")
    pltpu.sync_copy(b_hbm, b_v)
    bv = b_v[...]
    lane = lax.iota(jnp.int32, 16)

    def _fill(i, c):
        base = i * (16 * _FILL_UNROLL)
        for u in range(_FILL_UNROLL):
            row_v[pl.ds(base + u * 16, 16)] = bv
        return c

    lax.fori_loop(0, _FILL_STEPS, _fill, 0)

    for r in range(_ROWS_PER_W):
        row = wid * _ROWS_PER_W + r
        pltpu.sync_copy(key_hbm.at[row], key_v)
        pltpu.sync_copy(val_hbm.at[row], val_v)
        for d in range(_DEPTH):
            dbase = d * _NCHILD
            kbase = (row * _DEPTH + d) * _VOCAB
            raws = [key_v[pl.ds(dbase + c * 16, 16)] for c in range(4)]
            toks = [raws[c] - kbase for c in range(4)]
            cnts = [val_v[pl.ds(dbase + c * 16, 16)] for c in range(4)]
            if d == 0:
                olds = [bv] * 4
            else:
                olds = [plsc.load_gather(row_v, [toks[c]]) for c in range(4)]
            news = [olds[c] + cnts[c] for c in range(4)]
            for c in range(4):
                dup = lane < 0
                for j in range(1, 16):
                    bc = plsc.load_gather(
                        key_v, [jnp.full((16,), dbase + c * 16 + j, jnp.int32)])
                    dup = jnp.logical_or(
                        dup, jnp.logical_and(raws[c] == bc, lane < j))
                plsc.store_scatter(row_v, [toks[c]], news[c],
                                   mask=jnp.logical_not(dup))
        pltpu.sync_copy(row_v, out_hbm.at[row])
        for d in range(_DEPTH):
            kbase = (row * _DEPTH + d) * _VOCAB
            for c in range(4):
                tc = key_v[pl.ds(d * _NCHILD + c * 16, 16)] - kbase
                plsc.store_scatter(row_v, [tc], bv)


def _make_call():
    mesh = plsc.VectorSubcoreMesh(core_axis_name="c", subcore_axis_name="s")
    return pl.kernel(
        _sc_body,
        out_type=jax.ShapeDtypeStruct((_ROWS, _VOCAB), jnp.float32),
        mesh=mesh,
        compiler_params=pltpu.CompilerParams(needs_layout_passes=False),
        scratch_types=[
            pltpu.VMEM((_VOCAB,), jnp.float32),
            pltpu.VMEM((_UPD,), jnp.int32),
            pltpu.VMEM((_UPD,), jnp.float32),
            pltpu.VMEM((16,), jnp.float32),
        ],
    )


def kernel(idx, child_tokens, counts, W, b_lin):
    del idx
    tok4 = child_tokens.reshape(_ROWS, _DEPTH, _NCHILD)
    cnt4 = counts.reshape(_ROWS, _DEPTH, _NCHILD).astype(jnp.float32)
    block = jnp.arange(_ROWS * _DEPTH, dtype=jnp.int32).reshape(_ROWS, _DEPTH, 1)
    keys = (block * _VOCAB + tok4).reshape(-1)
    vals = (cnt4 * W.reshape(1, _DEPTH, 1).astype(jnp.float32)).reshape(-1)
    keys_s, vals_s = lax.sort((keys, vals), dimension=0,
                              num_keys=1, is_stable=False)
    key2 = keys_s.reshape(_ROWS, _UPD)
    val2 = vals_s.reshape(_ROWS, _UPD)
    bb = jnp.broadcast_to(b_lin.reshape(1).astype(jnp.float32), (16,))
    out = _make_call()(key2, val2, bb)
    return out.reshape(_B, _T, _VOCAB)
